# Initial kernel scaffold; baseline (speedup 1.0000x reference)
import jax
import jax.numpy as jnp
from jax.experimental import pallas as pl


def kernel(x, edge_index, i, j, W1, b1, W2, b2, Wlin, blin):
    raise NotImplementedError("write your pallas kernel here")


# speedup vs baseline: 8.6310x; 8.6310x over previous
import functools

import jax
import jax.numpy as jnp
from jax import lax
from jax.experimental import pallas as pl
from jax.experimental.pallas import tpu as pltpu
from jax.experimental.pallas import tpu_sc as plsc

N_NODES = 10000
D = 128
N_EDGES = 320000
N_PAIRS = 16384

NC = 2
NS = 16
L = 16
NW = NC * NS
NP = 10240
EB = 128
EPW = 10240
NB = EPW // EB
NE_PAD = NW * EPW
PAD_IDX = N_NODES + 100
RPT = NP // NS
PPW = N_PAIRS // NW
BLK = 1024

_mesh = plsc.VectorSubcoreMesh(core_axis_name="c", subcore_axis_name="s")


@functools.partial(
    pl.kernel,
    out_type=jax.ShapeDtypeStruct((NC, NP, L), jnp.float32),
    mesh=_mesh,
    scratch_types=[
        pltpu.VMEM_SHARED((NP, L), jnp.float32),
        pltpu.VMEM((NB, EB), jnp.int32),
        pltpu.VMEM((EB, L), jnp.float32),
        pltpu.VMEM((RPT, L), jnp.float32),
    ],
    compiler_params=pltpu.CompilerParams(use_tc_tiling_on_sc=False),
)
def _deg_kernel(dst_hbm, out_hbm, acc_sh, dst_v, ones_v, z_v):
    cid = lax.axis_index("c")
    sid = lax.axis_index("s")
    wid = cid * NS + sid

    zeros16 = jnp.zeros((L,), jnp.float32)
    ones16 = jnp.ones((L,), jnp.float32)

    @pl.loop(0, RPT)
    def _(r):
        z_v[r] = zeros16

    @pl.loop(0, EB)
    def _(r):
        ones_v[r] = ones16

    pltpu.sync_copy(z_v, acc_sh.at[pl.ds(sid * RPT, RPT)])
    pltpu.sync_copy(dst_hbm.at[wid], dst_v)
    plsc.subcore_barrier()

    @pl.loop(0, NB)
    def _(b):
        pltpu.sync_copy(ones_v, acc_sh.at[dst_v.at[b]], add=True)

    plsc.subcore_barrier()
    pltpu.sync_copy(acc_sh.at[pl.ds(sid * RPT, RPT)],
                    out_hbm.at[cid, pl.ds(sid * RPT, RPT)])


@functools.partial(
    pl.kernel,
    out_type=jax.ShapeDtypeStruct((NC, NP, D), jnp.float32),
    mesh=_mesh,
    scratch_types=[
        pltpu.VMEM_SHARED((NP, D), jnp.float32),
        pltpu.VMEM((NB, EB), jnp.int32),
        pltpu.VMEM((NB, EB), jnp.int32),
        pltpu.VMEM((EB, D), jnp.float32),
    ],
)
def _edge_kernel(g_hbm, src_hbm, dst_hbm, out_hbm,
                 acc_sh, src_v, dst_v, b0):
    cid = lax.axis_index("c")
    sid = lax.axis_index("s")
    wid = cid * NS + sid

    zeros16 = jnp.zeros((L,), jnp.float32)

    @pl.loop(0, EB)
    def _(r):
        for c in range(D // L):
            b0[r, pl.ds(c * L, L)] = zeros16

    @pl.loop(0, RPT // EB)
    def _(k):
        pltpu.sync_copy(b0, acc_sh.at[pl.ds(sid * RPT + k * EB, EB)])

    pltpu.sync_copy(src_hbm.at[wid], src_v)
    pltpu.sync_copy(dst_hbm.at[wid], dst_v)
    plsc.subcore_barrier()

    @pl.loop(0, NB)
    def _(b):
        pltpu.sync_copy(g_hbm.at[src_v.at[b]], b0)
        pltpu.sync_copy(b0, acc_sh.at[dst_v.at[b]], add=True)

    plsc.subcore_barrier()
    pltpu.sync_copy(acc_sh.at[pl.ds(sid * RPT, RPT)],
                    out_hbm.at[cid, pl.ds(sid * RPT, RPT)])


@functools.partial(
    pl.kernel,
    out_type=jax.ShapeDtypeStruct((NW, PPW), jnp.float32),
    mesh=_mesh,
    scratch_types=[
        pltpu.VMEM((NP,), jnp.float32),
        pltpu.VMEM((NP,), jnp.float32),
        pltpu.VMEM((PPW,), jnp.int32),
        pltpu.VMEM((PPW,), jnp.int32),
        pltpu.VMEM((PPW,), jnp.float32),
        pltpu.VMEM((L,), jnp.float32),
    ],
    compiler_params=pltpu.CompilerParams(needs_layout_passes=False),
)
def _pair_kernel(u_hbm, v_hbm, i_hbm, j_hbm, bl_hbm, out_hbm,
                 u_v, v_v, i_v, j_v, o_v, bl_v):
    cid = lax.axis_index("c")
    sid = lax.axis_index("s")
    wid = cid * NS + sid

    pltpu.sync_copy(u_hbm, u_v)
    pltpu.sync_copy(v_hbm, v_v)
    pltpu.sync_copy(i_hbm.at[wid], i_v)
    pltpu.sync_copy(j_hbm.at[wid], j_v)
    pltpu.sync_copy(bl_hbm, bl_v)
    bl = bl_v[...]

    @pl.loop(0, PPW // L)
    def _(k):
        ii = i_v[pl.ds(k * L, L)]
        jj = j_v[pl.ds(k * L, L)]
        a = plsc.load_gather(u_v, [ii])
        b = plsc.load_gather(v_v, [jj])
        s = a + b + bl
        o_v[pl.ds(k * L, L)] = 1.0 / (1.0 + jnp.exp(-s))

    pltpu.sync_copy(o_v, out_hbm.at[wid])


def _rsqrt_deg(dp0_ref, dp1_ref):
    deg = dp0_ref[:, 0:1] + dp1_ref[:, 0:1] + 1.0
    return lax.rsqrt(deg)


def _mm1_body(x_ref, w_ref, dp0_ref, dp1_ref, g_ref):
    r = _rsqrt_deg(dp0_ref, dp1_ref)
    h = jnp.dot(x_ref[...], w_ref[...], preferred_element_type=jnp.float32)
    g_ref[...] = r * h


def _mm2_body(sa_ref, sb_ref, g1_ref, dp0_ref, dp1_ref, w_ref, b_ref, g2_ref):
    r = _rsqrt_deg(dp0_ref, dp1_ref)
    t = sa_ref[...] + sb_ref[...] + g1_ref[...]
    x2 = jnp.maximum(r * t + b_ref[...], 0.0)
    h2 = jnp.dot(x2, w_ref[...], preferred_element_type=jnp.float32)
    g2_ref[...] = r * h2


def _fin_body(sa_ref, sb_ref, g2_ref, dp0_ref, dp1_ref, b_ref,
              wa_ref, wb_ref, u_ref, v_ref):
    r = _rsqrt_deg(dp0_ref, dp1_ref)
    h = r * (sa_ref[...] + sb_ref[...] + g2_ref[...]) + b_ref[...]
    u_ref[...] = jnp.sum(h * wa_ref[...], axis=1, keepdims=True)
    v_ref[...] = jnp.sum(h * wb_ref[...], axis=1, keepdims=True)


_row_spec = pl.BlockSpec((BLK, D), lambda m: (m, 0))
_dp_spec = pl.BlockSpec((BLK, L), lambda m: (m, 0))
_w_spec = pl.BlockSpec((D, D), lambda m: (0, 0))
_b_spec = pl.BlockSpec((1, D), lambda m: (0, 0))
_col_spec = pl.BlockSpec((BLK, 1), lambda m: (m, 0))

_mm1 = pl.pallas_call(
    _mm1_body,
    grid=(NP // BLK,),
    in_specs=[_row_spec, _w_spec, _dp_spec, _dp_spec],
    out_specs=_row_spec,
    out_shape=jax.ShapeDtypeStruct((NP, D), jnp.float32),
)

_mm2 = pl.pallas_call(
    _mm2_body,
    grid=(NP // BLK,),
    in_specs=[_row_spec, _row_spec, _row_spec, _dp_spec, _dp_spec,
              _w_spec, _b_spec],
    out_specs=_row_spec,
    out_shape=jax.ShapeDtypeStruct((NP, D), jnp.float32),
)

_fin = pl.pallas_call(
    _fin_body,
    grid=(NP // BLK,),
    in_specs=[_row_spec, _row_spec, _row_spec, _dp_spec, _dp_spec,
              _b_spec, _b_spec, _b_spec],
    out_specs=[_col_spec, _col_spec],
    out_shape=[jax.ShapeDtypeStruct((NP, 1), jnp.float32),
               jax.ShapeDtypeStruct((NP, 1), jnp.float32)],
)


def kernel(x, edge_index, i, j, W1, b1, W2, b2, Wlin, blin):
    src = edge_index[0].astype(jnp.int32)
    dst = edge_index[1].astype(jnp.int32)
    pad = jnp.full((NE_PAD - N_EDGES,), PAD_IDX, jnp.int32)
    src_p = jnp.concatenate([src, pad]).reshape(NW, NB, EB)
    dst_p = jnp.concatenate([dst, pad]).reshape(NW, NB, EB)
    x_p = jnp.pad(x, ((0, NP - N_NODES), (0, 0)))
    i_p = i.astype(jnp.int32).reshape(NW, PPW)
    j_p = j.astype(jnp.int32).reshape(NW, PPW)
    bl_p = jnp.broadcast_to(blin.astype(jnp.float32), (L,))

    degp = _deg_kernel(dst_p)
    dp0, dp1 = degp[0], degp[1]

    g1 = _mm1(x_p, W1, dp0, dp1)
    s1 = _edge_kernel(g1, src_p, dst_p)
    g2 = _mm2(s1[0], s1[1], g1, dp0, dp1, W2, b1.reshape(1, D))
    s2 = _edge_kernel(g2, src_p, dst_p)
    u, v = _fin(s2[0], s2[1], g2, dp0, dp1, b2.reshape(1, D),
                Wlin[:D, 0].reshape(1, D), Wlin[D:, 0].reshape(1, D))
    out = _pair_kernel(u.reshape(NP), v.reshape(NP), i_p, j_p, bl_p)
    return out.reshape(N_PAIRS, 1)


# speedup vs baseline: 9.2536x; 1.0721x over previous
import functools

import jax
import jax.numpy as jnp
from jax import lax
from jax.experimental import pallas as pl
from jax.experimental.pallas import tpu as pltpu
from jax.experimental.pallas import tpu_sc as plsc

N_NODES = 10000
D = 128
N_EDGES = 320000
N_PAIRS = 16384

NC = 2
NS = 16
L = 16
NW = NC * NS
NP = 10240
EB = 128
EPW = 10240
NB = EPW // EB
NE_PAD = NW * EPW
PAD_IDX = N_NODES + 100
RPT = NP // NS
PPW = N_PAIRS // NW
BLK = 1024

_mesh = plsc.VectorSubcoreMesh(core_axis_name="c", subcore_axis_name="s")


@functools.partial(
    pl.kernel,
    out_type=jax.ShapeDtypeStruct((NC, NP, L), jnp.float32),
    mesh=_mesh,
    scratch_types=[
        pltpu.VMEM_SHARED((NP, L), jnp.float32),
        pltpu.VMEM((NB, EB), jnp.int32),
        pltpu.VMEM((EB, L), jnp.float32),
        pltpu.VMEM((RPT, L), jnp.float32),
    ],
    compiler_params=pltpu.CompilerParams(use_tc_tiling_on_sc=False),
)
def _deg_kernel(dst_hbm, out_hbm, acc_sh, dst_v, ones_v, z_v):
    cid = lax.axis_index("c")
    sid = lax.axis_index("s")
    wid = cid * NS + sid

    zeros16 = jnp.zeros((L,), jnp.float32)
    ones16 = jnp.ones((L,), jnp.float32)

    @pl.loop(0, RPT)
    def _(r):
        z_v[r] = zeros16

    @pl.loop(0, EB)
    def _(r):
        ones_v[r] = ones16

    pltpu.sync_copy(z_v, acc_sh.at[pl.ds(sid * RPT, RPT)])
    pltpu.sync_copy(dst_hbm.at[wid], dst_v)
    plsc.subcore_barrier()

    @pl.loop(0, NB)
    def _(b):
        pltpu.sync_copy(ones_v, acc_sh.at[dst_v.at[b]], add=True)

    plsc.subcore_barrier()
    pltpu.sync_copy(acc_sh.at[pl.ds(sid * RPT, RPT)],
                    out_hbm.at[cid, pl.ds(sid * RPT, RPT)])


@functools.partial(
    pl.kernel,
    out_type=jax.ShapeDtypeStruct((NC, NP, D), jnp.float32),
    mesh=_mesh,
    scratch_types=[
        pltpu.VMEM_SHARED((NP, D), jnp.float32),
        pltpu.VMEM((NB // 2, EB), jnp.int32),
        pltpu.VMEM((NB // 2, EB), jnp.int32),
        pltpu.VMEM((EB, D), jnp.float32),
        pltpu.VMEM((EB, D), jnp.float32),
        pltpu.SemaphoreType.DMA,
        pltpu.SemaphoreType.DMA,
        pltpu.SemaphoreType.DMA,
        pltpu.SemaphoreType.DMA,
    ],
)
def _edge_kernel(g_hbm, src_hbm, dst_hbm, out_hbm,
                 acc_sh, src_v, dst_v, b0, b1, g0, g1, s0, s1):
    cid = lax.axis_index("c")
    sid = lax.axis_index("s")
    wid = cid * NS + sid

    zeros16 = jnp.zeros((L,), jnp.float32)

    @pl.loop(0, EB)
    def _(r):
        for c in range(D // L):
            b0[r, pl.ds(c * L, L)] = zeros16

    @pl.loop(0, RPT // EB)
    def _(k):
        pltpu.sync_copy(b0, acc_sh.at[pl.ds(sid * RPT + k * EB, EB)])

    plsc.subcore_barrier()

    NBH = NB // 2
    bufs = (b0, b1)
    gsem = (g0, g1)
    ssem = (s0, s1)
    for p in range(2):
        base = p * NBH
        pltpu.sync_copy(src_hbm.at[wid, pl.ds(base, NBH)], src_v)
        pltpu.sync_copy(dst_hbm.at[wid, pl.ds(base, NBH)], dst_v)
        pltpu.async_copy(g_hbm.at[src_v.at[0]], b0, g0)

        @pl.loop(0, NBH, step=2)
        def _(g):
            for kk in range(2):
                j = g + kk
                k, n = kk, 1 - kk
                pltpu.make_async_copy(
                    g_hbm.at[src_v.at[j]], bufs[k], gsem[k]).wait()
                pltpu.async_copy(
                    bufs[k], acc_sh.at[dst_v.at[j]], ssem[k], add=True)

                @pl.when(j >= 1)
                def _():
                    pltpu.make_async_copy(
                        bufs[n], acc_sh.at[dst_v.at[j]], ssem[n]).wait()

                @pl.when(j + 1 < NBH)
                def _():
                    pltpu.async_copy(
                        g_hbm.at[src_v.at[j + 1]], bufs[n], gsem[n])

        pltpu.make_async_copy(
            bufs[(NBH - 1) % 2], acc_sh.at[dst_v.at[0]],
            ssem[(NBH - 1) % 2]).wait()

    plsc.subcore_barrier()
    pltpu.sync_copy(acc_sh.at[pl.ds(sid * RPT, RPT)],
                    out_hbm.at[cid, pl.ds(sid * RPT, RPT)])


@functools.partial(
    pl.kernel,
    out_type=jax.ShapeDtypeStruct((NW, PPW), jnp.float32),
    mesh=_mesh,
    scratch_types=[
        pltpu.VMEM((NP,), jnp.float32),
        pltpu.VMEM((NP,), jnp.float32),
        pltpu.VMEM((PPW,), jnp.int32),
        pltpu.VMEM((PPW,), jnp.int32),
        pltpu.VMEM((PPW,), jnp.float32),
        pltpu.VMEM((L,), jnp.float32),
    ],
    compiler_params=pltpu.CompilerParams(needs_layout_passes=False),
)
def _pair_kernel(u_hbm, v_hbm, i_hbm, j_hbm, bl_hbm, out_hbm,
                 u_v, v_v, i_v, j_v, o_v, bl_v):
    cid = lax.axis_index("c")
    sid = lax.axis_index("s")
    wid = cid * NS + sid

    pltpu.sync_copy(u_hbm, u_v)
    pltpu.sync_copy(v_hbm, v_v)
    pltpu.sync_copy(i_hbm.at[wid], i_v)
    pltpu.sync_copy(j_hbm.at[wid], j_v)
    pltpu.sync_copy(bl_hbm, bl_v)
    bl = bl_v[...]

    @pl.loop(0, PPW // L)
    def _(k):
        ii = i_v[pl.ds(k * L, L)]
        jj = j_v[pl.ds(k * L, L)]
        a = plsc.load_gather(u_v, [ii])
        b = plsc.load_gather(v_v, [jj])
        s = a + b + bl
        o_v[pl.ds(k * L, L)] = 1.0 / (1.0 + jnp.exp(-s))

    pltpu.sync_copy(o_v, out_hbm.at[wid])


def _rsqrt_deg(dp0_ref, dp1_ref):
    deg = dp0_ref[:, 0:1] + dp1_ref[:, 0:1] + 1.0
    return lax.rsqrt(deg)


def _mm1_body(x_ref, w_ref, dp0_ref, dp1_ref, g_ref):
    r = _rsqrt_deg(dp0_ref, dp1_ref)
    h = jnp.dot(x_ref[...], w_ref[...], preferred_element_type=jnp.float32)
    g_ref[...] = r * h


def _mm2_body(sa_ref, sb_ref, g1_ref, dp0_ref, dp1_ref, w_ref, b_ref, g2_ref):
    r = _rsqrt_deg(dp0_ref, dp1_ref)
    t = sa_ref[...] + sb_ref[...] + g1_ref[...]
    x2 = jnp.maximum(r * t + b_ref[...], 0.0)
    h2 = jnp.dot(x2, w_ref[...], preferred_element_type=jnp.float32)
    g2_ref[...] = r * h2


def _fin_body(sa_ref, sb_ref, g2_ref, dp0_ref, dp1_ref, b_ref,
              wa_ref, wb_ref, u_ref, v_ref):
    r = _rsqrt_deg(dp0_ref, dp1_ref)
    h = r * (sa_ref[...] + sb_ref[...] + g2_ref[...]) + b_ref[...]
    u_ref[...] = jnp.sum(h * wa_ref[...], axis=1, keepdims=True)
    v_ref[...] = jnp.sum(h * wb_ref[...], axis=1, keepdims=True)


_row_spec = pl.BlockSpec((BLK, D), lambda m: (m, 0))
_dp_spec = pl.BlockSpec((BLK, L), lambda m: (m, 0))
_w_spec = pl.BlockSpec((D, D), lambda m: (0, 0))
_b_spec = pl.BlockSpec((1, D), lambda m: (0, 0))
_col_spec = pl.BlockSpec((BLK, 1), lambda m: (m, 0))

_mm1 = pl.pallas_call(
    _mm1_body,
    grid=(NP // BLK,),
    in_specs=[_row_spec, _w_spec, _dp_spec, _dp_spec],
    out_specs=_row_spec,
    out_shape=jax.ShapeDtypeStruct((NP, D), jnp.float32),
)

_mm2 = pl.pallas_call(
    _mm2_body,
    grid=(NP // BLK,),
    in_specs=[_row_spec, _row_spec, _row_spec, _dp_spec, _dp_spec,
              _w_spec, _b_spec],
    out_specs=_row_spec,
    out_shape=jax.ShapeDtypeStruct((NP, D), jnp.float32),
)

_fin = pl.pallas_call(
    _fin_body,
    grid=(NP // BLK,),
    in_specs=[_row_spec, _row_spec, _row_spec, _dp_spec, _dp_spec,
              _b_spec, _b_spec, _b_spec],
    out_specs=[_col_spec, _col_spec],
    out_shape=[jax.ShapeDtypeStruct((NP, 1), jnp.float32),
               jax.ShapeDtypeStruct((NP, 1), jnp.float32)],
)


def kernel(x, edge_index, i, j, W1, b1, W2, b2, Wlin, blin):
    src = edge_index[0].astype(jnp.int32)
    dst = edge_index[1].astype(jnp.int32)
    pad = jnp.full((NE_PAD - N_EDGES,), PAD_IDX, jnp.int32)
    src_p = jnp.concatenate([src, pad]).reshape(NW, NB, EB)
    dst_p = jnp.concatenate([dst, pad]).reshape(NW, NB, EB)
    x_p = jnp.pad(x, ((0, NP - N_NODES), (0, 0)))
    i_p = i.astype(jnp.int32).reshape(NW, PPW)
    j_p = j.astype(jnp.int32).reshape(NW, PPW)
    bl_p = jnp.broadcast_to(blin.astype(jnp.float32), (L,))

    degp = _deg_kernel(dst_p)
    dp0, dp1 = degp[0], degp[1]

    g1 = _mm1(x_p, W1, dp0, dp1)
    s1 = _edge_kernel(g1, src_p, dst_p)
    g2 = _mm2(s1[0], s1[1], g1, dp0, dp1, W2, b1.reshape(1, D))
    s2 = _edge_kernel(g2, src_p, dst_p)
    u, v = _fin(s2[0], s2[1], g2, dp0, dp1, b2.reshape(1, D),
                Wlin[:D, 0].reshape(1, D), Wlin[D:, 0].reshape(1, D))
    out = _pair_kernel(u.reshape(NP), v.reshape(NP), i_p, j_p, bl_p)
    return out.reshape(N_PAIRS, 1)


# speedup vs baseline: 9.2550x; 1.0002x over previous
import functools

import jax
import jax.numpy as jnp
from jax import lax
from jax.experimental import pallas as pl
from jax.experimental.pallas import tpu as pltpu
from jax.experimental.pallas import tpu_sc as plsc

N_NODES = 10000
D = 128
N_EDGES = 320000
N_PAIRS = 16384

NC = 2
NS = 16
L = 16
NW = NC * NS
NP = 10240
EB = 128
EPW = 10240
NB = EPW // EB
NE_PAD = NW * EPW
PAD_IDX = N_NODES + 100
RPT = NP // NS
PPW = N_PAIRS // NW
BLK = 1024

_mesh = plsc.VectorSubcoreMesh(core_axis_name="c", subcore_axis_name="s")


@functools.partial(
    pl.kernel,
    out_type=jax.ShapeDtypeStruct((NC, NP, L), jnp.float32),
    mesh=_mesh,
    scratch_types=[
        pltpu.VMEM_SHARED((NP, L), jnp.float32),
        pltpu.VMEM((NB, EB), jnp.int32),
        pltpu.VMEM((EB, L), jnp.float32),
        pltpu.VMEM((RPT, L), jnp.float32),
    ],
    compiler_params=pltpu.CompilerParams(use_tc_tiling_on_sc=False),
)
def _deg_kernel(dst_hbm, out_hbm, acc_sh, dst_v, ones_v, z_v):
    cid = lax.axis_index("c")
    sid = lax.axis_index("s")
    wid = cid * NS + sid

    zeros16 = jnp.zeros((L,), jnp.float32)
    ones16 = jnp.ones((L,), jnp.float32)

    @pl.loop(0, RPT)
    def _(r):
        z_v[r] = zeros16

    @pl.loop(0, EB)
    def _(r):
        ones_v[r] = ones16

    pltpu.sync_copy(z_v, acc_sh.at[pl.ds(sid * RPT, RPT)])
    pltpu.sync_copy(dst_hbm.at[wid], dst_v)
    plsc.subcore_barrier()

    @pl.loop(0, NB)
    def _(b):
        pltpu.sync_copy(ones_v, acc_sh.at[dst_v.at[b]], add=True)

    plsc.subcore_barrier()
    pltpu.sync_copy(acc_sh.at[pl.ds(sid * RPT, RPT)],
                    out_hbm.at[cid, pl.ds(sid * RPT, RPT)])


@functools.partial(
    pl.kernel,
    out_type=jax.ShapeDtypeStruct((NC, NP, D), jnp.float32),
    mesh=_mesh,
    scratch_types=[
        pltpu.VMEM_SHARED((NP, D), jnp.float32),
        pltpu.VMEM((NB // 2, EB), jnp.int32),
        pltpu.VMEM((NB // 2, EB), jnp.int32),
        pltpu.VMEM((EB, D), jnp.float32),
        pltpu.VMEM((EB, D), jnp.float32),
        pltpu.SemaphoreType.DMA,
        pltpu.SemaphoreType.DMA,
        pltpu.SemaphoreType.DMA,
        pltpu.SemaphoreType.DMA,
    ],
)
def _edge_kernel(g_hbm, src_hbm, dst_hbm, out_hbm,
                 acc_sh, src_v, dst_v, b0, b1, g0, g1, s0, s1):
    cid = lax.axis_index("c")
    sid = lax.axis_index("s")
    wid = cid * NS + sid

    zeros16 = jnp.zeros((L,), jnp.float32)

    @pl.loop(0, EB)
    def _(r):
        for c in range(D // L):
            b0[r, pl.ds(c * L, L)] = zeros16

    @pl.loop(0, RPT // EB)
    def _(k):
        pltpu.sync_copy(b0, acc_sh.at[pl.ds(sid * RPT + k * EB, EB)])

    plsc.subcore_barrier()

    NBH = NB // 2
    bufs = (b0, b1)
    gsem = (g0, g1)
    ssem = (s0, s1)

    def _start_gather(j, buf, sem):
        pltpu.async_copy(g_hbm.at[src_v.at[j, pl.ds(0, EB // 2)]],
                         buf.at[pl.ds(0, EB // 2)], sem)
        pltpu.async_copy(g_hbm.at[src_v.at[j, pl.ds(EB // 2, EB // 2)]],
                         buf.at[pl.ds(EB // 2, EB // 2)], sem)

    for p in range(2):
        base = p * NBH
        pltpu.sync_copy(src_hbm.at[wid, pl.ds(base, NBH)], src_v)
        pltpu.sync_copy(dst_hbm.at[wid, pl.ds(base, NBH)], dst_v)
        _start_gather(0, b0, g0)

        @pl.loop(0, NBH, step=2)
        def _(g):
            for kk in range(2):
                j = g + kk
                k, n = kk, 1 - kk
                pltpu.make_async_copy(
                    g_hbm.at[src_v.at[j]], bufs[k], gsem[k]).wait()
                pltpu.async_copy(
                    bufs[k], acc_sh.at[dst_v.at[j]], ssem[k], add=True)

                @pl.when(j >= 1)
                def _():
                    pltpu.make_async_copy(
                        bufs[n], acc_sh.at[dst_v.at[j]], ssem[n]).wait()

                @pl.when(j + 1 < NBH)
                def _():
                    _start_gather(j + 1, bufs[n], gsem[n])

        pltpu.make_async_copy(
            bufs[(NBH - 1) % 2], acc_sh.at[dst_v.at[0]],
            ssem[(NBH - 1) % 2]).wait()

    plsc.subcore_barrier()
    pltpu.sync_copy(acc_sh.at[pl.ds(sid * RPT, RPT)],
                    out_hbm.at[cid, pl.ds(sid * RPT, RPT)])


@functools.partial(
    pl.kernel,
    out_type=jax.ShapeDtypeStruct((NW, PPW), jnp.float32),
    mesh=_mesh,
    scratch_types=[
        pltpu.VMEM((NP,), jnp.float32),
        pltpu.VMEM((NP,), jnp.float32),
        pltpu.VMEM((PPW,), jnp.int32),
        pltpu.VMEM((PPW,), jnp.int32),
        pltpu.VMEM((PPW,), jnp.float32),
        pltpu.VMEM((L,), jnp.float32),
    ],
    compiler_params=pltpu.CompilerParams(needs_layout_passes=False),
)
def _pair_kernel(u_hbm, v_hbm, i_hbm, j_hbm, bl_hbm, out_hbm,
                 u_v, v_v, i_v, j_v, o_v, bl_v):
    cid = lax.axis_index("c")
    sid = lax.axis_index("s")
    wid = cid * NS + sid

    pltpu.sync_copy(u_hbm, u_v)
    pltpu.sync_copy(v_hbm, v_v)
    pltpu.sync_copy(i_hbm.at[wid], i_v)
    pltpu.sync_copy(j_hbm.at[wid], j_v)
    pltpu.sync_copy(bl_hbm, bl_v)
    bl = bl_v[...]

    @pl.loop(0, PPW // L)
    def _(k):
        ii = i_v[pl.ds(k * L, L)]
        jj = j_v[pl.ds(k * L, L)]
        a = plsc.load_gather(u_v, [ii])
        b = plsc.load_gather(v_v, [jj])
        s = a + b + bl
        o_v[pl.ds(k * L, L)] = 1.0 / (1.0 + jnp.exp(-s))

    pltpu.sync_copy(o_v, out_hbm.at[wid])


def _rsqrt_deg(dp0_ref, dp1_ref):
    deg = dp0_ref[:, 0:1] + dp1_ref[:, 0:1] + 1.0
    return lax.rsqrt(deg)


def _mm1_body(x_ref, w_ref, dp0_ref, dp1_ref, g_ref):
    r = _rsqrt_deg(dp0_ref, dp1_ref)
    h = jnp.dot(x_ref[...], w_ref[...], preferred_element_type=jnp.float32)
    g_ref[...] = r * h


def _mm2_body(sa_ref, sb_ref, g1_ref, dp0_ref, dp1_ref, w_ref, b_ref, g2_ref):
    r = _rsqrt_deg(dp0_ref, dp1_ref)
    t = sa_ref[...] + sb_ref[...] + g1_ref[...]
    x2 = jnp.maximum(r * t + b_ref[...], 0.0)
    h2 = jnp.dot(x2, w_ref[...], preferred_element_type=jnp.float32)
    g2_ref[...] = r * h2


def _fin_body(sa_ref, sb_ref, g2_ref, dp0_ref, dp1_ref, b_ref,
              wa_ref, wb_ref, u_ref, v_ref):
    r = _rsqrt_deg(dp0_ref, dp1_ref)
    h = r * (sa_ref[...] + sb_ref[...] + g2_ref[...]) + b_ref[...]
    u_ref[...] = jnp.sum(h * wa_ref[...], axis=1, keepdims=True)
    v_ref[...] = jnp.sum(h * wb_ref[...], axis=1, keepdims=True)


_row_spec = pl.BlockSpec((BLK, D), lambda m: (m, 0))
_dp_spec = pl.BlockSpec((BLK, L), lambda m: (m, 0))
_w_spec = pl.BlockSpec((D, D), lambda m: (0, 0))
_b_spec = pl.BlockSpec((1, D), lambda m: (0, 0))
_col_spec = pl.BlockSpec((BLK, 1), lambda m: (m, 0))

_mm1 = pl.pallas_call(
    _mm1_body,
    grid=(NP // BLK,),
    in_specs=[_row_spec, _w_spec, _dp_spec, _dp_spec],
    out_specs=_row_spec,
    out_shape=jax.ShapeDtypeStruct((NP, D), jnp.float32),
)

_mm2 = pl.pallas_call(
    _mm2_body,
    grid=(NP // BLK,),
    in_specs=[_row_spec, _row_spec, _row_spec, _dp_spec, _dp_spec,
              _w_spec, _b_spec],
    out_specs=_row_spec,
    out_shape=jax.ShapeDtypeStruct((NP, D), jnp.float32),
)

_fin = pl.pallas_call(
    _fin_body,
    grid=(NP // BLK,),
    in_specs=[_row_spec, _row_spec, _row_spec, _dp_spec, _dp_spec,
              _b_spec, _b_spec, _b_spec],
    out_specs=[_col_spec, _col_spec],
    out_shape=[jax.ShapeDtypeStruct((NP, 1), jnp.float32),
               jax.ShapeDtypeStruct((NP, 1), jnp.float32)],
)


def kernel(x, edge_index, i, j, W1, b1, W2, b2, Wlin, blin):
    src = edge_index[0].astype(jnp.int32)
    dst = edge_index[1].astype(jnp.int32)
    pad = jnp.full((NE_PAD - N_EDGES,), PAD_IDX, jnp.int32)
    src_p = jnp.concatenate([src, pad]).reshape(NW, NB, EB)
    dst_p = jnp.concatenate([dst, pad]).reshape(NW, NB, EB)
    x_p = jnp.pad(x, ((0, NP - N_NODES), (0, 0)))
    i_p = i.astype(jnp.int32).reshape(NW, PPW)
    j_p = j.astype(jnp.int32).reshape(NW, PPW)
    bl_p = jnp.broadcast_to(blin.astype(jnp.float32), (L,))

    degp = _deg_kernel(dst_p)
    dp0, dp1 = degp[0], degp[1]

    g1 = _mm1(x_p, W1, dp0, dp1)
    s1 = _edge_kernel(g1, src_p, dst_p)
    g2 = _mm2(s1[0], s1[1], g1, dp0, dp1, W2, b1.reshape(1, D))
    s2 = _edge_kernel(g2, src_p, dst_p)
    u, v = _fin(s2[0], s2[1], g2, dp0, dp1, b2.reshape(1, D),
                Wlin[:D, 0].reshape(1, D), Wlin[D:, 0].reshape(1, D))
    out = _pair_kernel(u.reshape(NP), v.reshape(NP), i_p, j_p, bl_p)
    return out.reshape(N_PAIRS, 1)
